# Initial kernel scaffold; baseline (speedup 1.0000x reference)
import jax
import jax.numpy as jnp
from jax.experimental import pallas as pl


def kernel(x, W1, b1, W2, b2, W3, b3, W4, b4, W5, b5, W6, b6):
    raise NotImplementedError("write your pallas kernel here")


# speedup vs baseline: 13.2734x; 13.2734x over previous
import functools

import jax
import jax.numpy as jnp
from jax import lax
from jax.experimental import pallas as pl
from jax.experimental.pallas import tpu as pltpu
from jax.experimental.pallas import tpu_sc as plsc

_K = 16
_NCLS = 8
_R = 256
_RM = 512
_NC = 2
_NS = 16
_NW = _NC * _NS
_P = 8


def _knn_conv1_call(x, xt, wd1, wb1, b1):
    B, N, _ = x.shape
    nb = N // _R

    def body(xr_ref, xt_ref, wd_ref, wb_ref, b1_ref, idx_ref, c1_ref, g1_ref):
        b = pl.program_id(0)
        r = pl.program_id(1)
        xr = xr_ref[0]
        xa = xt_ref[0]
        sq_r = jnp.sum(xr * xr, axis=1)
        sq_a = jnp.sum(xa * xa, axis=0)
        dot = jnp.dot(xr, xa)
        dist = sq_r[:, None] + sq_a[None, :] - 2.0 * dot
        cols = lax.broadcasted_iota(jnp.int32, (_R, N), 1)
        rows_g = r * _R + lax.broadcasted_iota(jnp.int32, (_R, N), 0)
        dist = jnp.where(cols == rows_g, dist + jnp.float32(1e10), dist)
        picks = []
        for _t in range(_K):
            m = jnp.min(dist, axis=1, keepdims=True)
            cand = jnp.where(dist == m, cols, jnp.int32(2**30))
            am = jnp.min(cand, axis=1, keepdims=True)
            picks.append(am)
            dist = jnp.where(cols == am, jnp.float32(2e10), dist)
        idx_ref[0] = jnp.concatenate(picks, axis=1) + b * N
        c1_ref[0] = jnp.dot(xr, wd_ref[...]) + b1_ref[...]
        g1_ref[0] = jnp.concatenate(
            [jnp.dot(xr, wb_ref[...]), jnp.zeros((_R, 64), jnp.float32)], axis=1)

    return pl.pallas_call(
        body,
        grid=(B, nb),
        in_specs=[
            pl.BlockSpec((1, _R, 3), lambda b, r: (b, r, 0)),
            pl.BlockSpec((1, 3, N), lambda b, r: (b, 0, 0)),
            pl.BlockSpec((3, 64), lambda b, r: (0, 0)),
            pl.BlockSpec((3, 64), lambda b, r: (0, 0)),
            pl.BlockSpec((1, 64), lambda b, r: (0, 0)),
        ],
        out_specs=[
            pl.BlockSpec((1, _R, _K), lambda b, r: (b, r, 0)),
            pl.BlockSpec((1, _R, 64), lambda b, r: (b, r, 0)),
            pl.BlockSpec((1, _R, 128), lambda b, r: (b, r, 0)),
        ],
        out_shape=[
            jax.ShapeDtypeStruct((B, N, _K), jnp.int32),
            jax.ShapeDtypeStruct((B, N, 64), jnp.float32),
            jax.ShapeDtypeStruct((B, N, 128), jnp.float32),
        ],
    )(x, xt, wd1, wb1, b1)


def _gather_max(g, idx_flat):
    M, C = g.shape
    pw = M // _NW
    nchunks = pw // _P
    mesh = plsc.VectorSubcoreMesh(core_axis_name="c", subcore_axis_name="s")

    @functools.partial(
        pl.kernel,
        mesh=mesh,
        out_type=jax.ShapeDtypeStruct((M, C), jnp.float32),
        scratch_types=[
            pltpu.VMEM((_P * _K,), jnp.int32),
            pltpu.VMEM((_P * _K, C), jnp.float32),
            pltpu.VMEM((_P, C), jnp.float32),
            pltpu.SemaphoreType.DMA,
        ],
    )
    def k(g_hbm, idx_hbm, out_hbm, idx_v, rows_v, out_v, sem):
        wid = lax.axis_index("s") * _NC + lax.axis_index("c")
        base = wid * pw

        def chunk(ci, carry):
            p0 = base + ci * _P
            pltpu.sync_copy(idx_hbm.at[pl.ds(p0 * _K, _P * _K)], idx_v)
            pltpu.async_copy(g_hbm.at[idx_v], rows_v, sem).wait()
            for p in range(_P):
                for ch in range(C // 16):
                    acc = rows_v[p * _K, pl.ds(ch * 16, 16)]
                    for j in range(1, _K):
                        acc = jnp.maximum(acc, rows_v[p * _K + j, pl.ds(ch * 16, 16)])
                    out_v[p, pl.ds(ch * 16, 16)] = acc
            pltpu.sync_copy(out_v, out_hbm.at[pl.ds(p0, _P)])
            return carry

        lax.fori_loop(0, nchunks, chunk, 0)

    return k(g, idx_flat)


def _conv2_linears(c1, gm1, wd2, wb2, b2):
    M = c1.shape[0]

    def body(c1_ref, gm_ref, wd_ref, wb_ref, b2_ref, f1_ref, c2_ref, g2_ref):
        f1 = jnp.maximum(c1_ref[...] + gm_ref[...][:, :64], 0.0)
        f1_ref[...] = f1
        c2_ref[...] = jnp.dot(f1, wd_ref[...]) + b2_ref[...]
        g2_ref[...] = jnp.dot(f1, wb_ref[...])

    return pl.pallas_call(
        body,
        grid=(M // _RM,),
        in_specs=[
            pl.BlockSpec((_RM, 64), lambda i: (i, 0)),
            pl.BlockSpec((_RM, 128), lambda i: (i, 0)),
            pl.BlockSpec((64, 128), lambda i: (0, 0)),
            pl.BlockSpec((64, 128), lambda i: (0, 0)),
            pl.BlockSpec((1, 128), lambda i: (0, 0)),
        ],
        out_specs=[
            pl.BlockSpec((_RM, 64), lambda i: (i, 0)),
            pl.BlockSpec((_RM, 128), lambda i: (i, 0)),
            pl.BlockSpec((_RM, 128), lambda i: (i, 0)),
        ],
        out_shape=[
            jax.ShapeDtypeStruct((M, 64), jnp.float32),
            jax.ShapeDtypeStruct((M, 128), jnp.float32),
            jax.ShapeDtypeStruct((M, 128), jnp.float32),
        ],
    )(c1, gm1, wd2, wb2, b2)


def _head_pool(f1, c2, gm2, w3a, w3b, b3):
    B, N, _ = f1.shape

    def body(f1_ref, c2_ref, gm_ref, wa_ref, wb_ref, b3_ref, out_ref):
        f2 = jnp.maximum(c2_ref[0] + gm_ref[0], 0.0)
        pf = jnp.maximum(
            jnp.dot(f1_ref[0], wa_ref[...]) + jnp.dot(f2, wb_ref[...]) + b3_ref[...],
            0.0,
        )
        bmax = jnp.max(pf, axis=0, keepdims=True)

        @pl.when(pl.program_id(1) == 0)
        def _():
            out_ref[0] = bmax

        @pl.when(pl.program_id(1) > 0)
        def _():
            out_ref[0] = jnp.maximum(out_ref[0], bmax)

    return pl.pallas_call(
        body,
        grid=(B, N // _RM),
        in_specs=[
            pl.BlockSpec((1, _RM, 64), lambda b, r: (b, r, 0)),
            pl.BlockSpec((1, _RM, 128), lambda b, r: (b, r, 0)),
            pl.BlockSpec((1, _RM, 128), lambda b, r: (b, r, 0)),
            pl.BlockSpec((64, 256), lambda b, r: (0, 0)),
            pl.BlockSpec((128, 256), lambda b, r: (0, 0)),
            pl.BlockSpec((1, 256), lambda b, r: (0, 0)),
        ],
        out_specs=pl.BlockSpec((1, 1, 256), lambda b, r: (b, 0, 0)),
        out_shape=jax.ShapeDtypeStruct((B, 1, 256), jnp.float32),
    )(f1, c2, gm2, w3a, w3b, b3).reshape(B, 256)


def _decoder(pooled, w4, b4, w5, b5, w6, b6):
    B = pooled.shape[0]

    def body(p_ref, w4_ref, b4_ref, w5_ref, b5_ref, w6_ref, b6_ref,
             lab_ref, conf_ref):
        h = jnp.maximum(jnp.dot(p_ref[...], w4_ref[...]) + b4_ref[...], 0.0)
        h = jnp.maximum(jnp.dot(h, w5_ref[...]) + b5_ref[...], 0.0)
        logits = jnp.dot(h, w6_ref[...]) + b6_ref[...]
        m = jnp.max(logits, axis=1, keepdims=True)
        cls = lax.broadcasted_iota(jnp.int32, logits.shape, 1)
        lab_ref[...] = jnp.min(
            jnp.where(logits == m, cls, jnp.int32(_NCLS)), axis=1, keepdims=True)
        conf_ref[...] = 1.0 / jnp.sum(jnp.exp(logits - m), axis=1, keepdims=True)

    return pl.pallas_call(
        body,
        out_shape=[
            jax.ShapeDtypeStruct((B, 1), jnp.int32),
            jax.ShapeDtypeStruct((B, 1), jnp.float32),
        ],
    )(pooled, w4, b4, w5, b5, w6, b6)


def kernel(x, W1, b1, W2, b2, W3, b3, W4, b4, W5, b5, W6, b6):
    B, N, _ = x.shape
    M = B * N
    xt = jnp.transpose(x, (0, 2, 1))
    idx, c1, g1 = _knn_conv1_call(
        x, xt, W1[:3] - W1[3:], W1[3:], b1.reshape(1, -1))
    idx_flat = idx.reshape(M * _K)
    gm1 = _gather_max(g1.reshape(M, 128), idx_flat)
    f1, c2, g2 = _conv2_linears(
        c1.reshape(M, 64), gm1, W2[:64] - W2[64:], W2[64:], b2.reshape(1, -1))
    gm2 = _gather_max(g2, idx_flat)
    pooled = _head_pool(
        f1.reshape(B, N, 64), c2.reshape(B, N, 128), gm2.reshape(B, N, 128),
        W3[:64], W3[64:], b3.reshape(1, -1))
    lab, conf = _decoder(
        pooled, W4, b4.reshape(1, -1), W5, b5.reshape(1, -1), W6,
        b6.reshape(1, -1))
    labels = jnp.broadcast_to(lab, (B, N))
    confidences = jnp.broadcast_to(conf, (B, N))
    return labels, confidences


# speedup vs baseline: 14.9637x; 1.1273x over previous
import functools

import jax
import jax.numpy as jnp
from jax import lax
from jax.experimental import pallas as pl
from jax.experimental.pallas import tpu as pltpu
from jax.experimental.pallas import tpu_sc as plsc

_K = 16
_NCLS = 8
_R = 256
_RM = 512
_NC = 2
_NS = 16
_NW = _NC * _NS
_P = 8


def _knn_conv1_call(x, xt, wd1, wb1, b1):
    B, N, _ = x.shape
    nb = N // _R

    def body(xr_ref, xt_ref, wd_ref, wb_ref, b1_ref, idx_ref, c1_ref, g1_ref):
        b = pl.program_id(0)
        r = pl.program_id(1)
        xr = xr_ref[0]
        xa = xt_ref[0]
        sq_r = jnp.sum(xr * xr, axis=1)
        sq_a = jnp.sum(xa * xa, axis=0)
        dot = jnp.dot(xr, xa)
        dist = sq_r[:, None] + sq_a[None, :] - 2.0 * dot
        cols = lax.broadcasted_iota(jnp.int32, (_R, N), 1)
        rows_g = r * _R + lax.broadcasted_iota(jnp.int32, (_R, N), 0)
        dist = jnp.where(cols == rows_g, dist + jnp.float32(1e10), dist)
        picks = []
        for _t in range(_K):
            m = jnp.min(dist, axis=1, keepdims=True)
            cand = jnp.where(dist == m, cols, jnp.int32(2**30))
            am = jnp.min(cand, axis=1, keepdims=True)
            picks.append(am)
            dist = jnp.where(cols == am, jnp.float32(2e10), dist)
        idx_ref[0] = jnp.concatenate(picks, axis=1) + b * N
        c1_ref[0] = jnp.dot(xr, wd_ref[...]) + b1_ref[...]
        g1_ref[0] = jnp.concatenate(
            [jnp.dot(xr, wb_ref[...]), jnp.zeros((_R, 64), jnp.float32)], axis=1)

    return pl.pallas_call(
        body,
        grid=(B, nb),
        in_specs=[
            pl.BlockSpec((1, _R, 3), lambda b, r: (b, r, 0)),
            pl.BlockSpec((1, 3, N), lambda b, r: (b, 0, 0)),
            pl.BlockSpec((3, 64), lambda b, r: (0, 0)),
            pl.BlockSpec((3, 64), lambda b, r: (0, 0)),
            pl.BlockSpec((1, 64), lambda b, r: (0, 0)),
        ],
        out_specs=[
            pl.BlockSpec((1, _R, _K), lambda b, r: (b, r, 0)),
            pl.BlockSpec((1, _R, 64), lambda b, r: (b, r, 0)),
            pl.BlockSpec((1, _R, 128), lambda b, r: (b, r, 0)),
        ],
        out_shape=[
            jax.ShapeDtypeStruct((B, N, _K), jnp.int32),
            jax.ShapeDtypeStruct((B, N, 64), jnp.float32),
            jax.ShapeDtypeStruct((B, N, 128), jnp.float32),
        ],
    )(x, xt, wd1, wb1, b1)


def _gather_max(g, idx_flat):
    M, C = g.shape
    pw = M // _NW
    nchunks = pw // _P
    mesh = plsc.VectorSubcoreMesh(core_axis_name="c", subcore_axis_name="s")

    ch_idx = _P * _K

    @functools.partial(
        pl.kernel,
        mesh=mesh,
        out_type=jax.ShapeDtypeStruct((M, C), jnp.float32),
        scratch_types=[
            pltpu.VMEM((pw * _K,), jnp.int32),
            pltpu.VMEM((ch_idx, C), jnp.float32),
            pltpu.VMEM((ch_idx, C), jnp.float32),
            pltpu.VMEM((_P, C), jnp.float32),
            pltpu.SemaphoreType.DMA,
            pltpu.SemaphoreType.DMA,
        ],
    )
    def k(g_hbm, idx_hbm, out_hbm, idx_all, rows0, rows1, out_v, sem0, sem1):
        wid = lax.axis_index("s") * _NC + lax.axis_index("c")
        base = wid * pw
        pltpu.sync_copy(idx_hbm.at[pl.ds(base * _K, pw * _K)], idx_all)

        def fire(ci, rows_v, sem):
            pltpu.async_copy(
                g_hbm.at[idx_all.at[pl.ds(ci * ch_idx, ch_idx)]], rows_v, sem)

        def drain(ci, rows_v, sem):
            pltpu.make_async_copy(
                g_hbm.at[idx_all.at[pl.ds(ci * ch_idx, ch_idx)]], rows_v,
                sem).wait()

        def compute(ci, rows_v):
            for p in range(_P):
                for ch in range(C // 16):
                    acc = rows_v[p * _K, pl.ds(ch * 16, 16)]
                    for j in range(1, _K):
                        acc = jnp.maximum(acc, rows_v[p * _K + j, pl.ds(ch * 16, 16)])
                    out_v[p, pl.ds(ch * 16, 16)] = acc
            pltpu.sync_copy(out_v, out_hbm.at[pl.ds(base + ci * _P, _P)])

        fire(0, rows0, sem0)

        def body(i, carry):
            e = 2 * i
            fire(e + 1, rows1, sem1)
            drain(e, rows0, sem0)
            compute(e, rows0)

            @pl.when(i + 1 < nchunks // 2)
            def _():
                fire(e + 2, rows0, sem0)

            drain(e + 1, rows1, sem1)
            compute(e + 1, rows1)
            return carry

        lax.fori_loop(0, nchunks // 2, body, 0)

    return k(g, idx_flat)


def _conv2_linears(c1, gm1, wd2, wb2, b2):
    M = c1.shape[0]

    def body(c1_ref, gm_ref, wd_ref, wb_ref, b2_ref, f1_ref, c2_ref, g2_ref):
        f1 = jnp.maximum(c1_ref[...] + gm_ref[...][:, :64], 0.0)
        f1_ref[...] = f1
        c2_ref[...] = jnp.dot(f1, wd_ref[...]) + b2_ref[...]
        g2_ref[...] = jnp.dot(f1, wb_ref[...])

    return pl.pallas_call(
        body,
        grid=(M // _RM,),
        in_specs=[
            pl.BlockSpec((_RM, 64), lambda i: (i, 0)),
            pl.BlockSpec((_RM, 128), lambda i: (i, 0)),
            pl.BlockSpec((64, 128), lambda i: (0, 0)),
            pl.BlockSpec((64, 128), lambda i: (0, 0)),
            pl.BlockSpec((1, 128), lambda i: (0, 0)),
        ],
        out_specs=[
            pl.BlockSpec((_RM, 64), lambda i: (i, 0)),
            pl.BlockSpec((_RM, 128), lambda i: (i, 0)),
            pl.BlockSpec((_RM, 128), lambda i: (i, 0)),
        ],
        out_shape=[
            jax.ShapeDtypeStruct((M, 64), jnp.float32),
            jax.ShapeDtypeStruct((M, 128), jnp.float32),
            jax.ShapeDtypeStruct((M, 128), jnp.float32),
        ],
    )(c1, gm1, wd2, wb2, b2)


def _head_pool(f1, c2, gm2, w3a, w3b, b3):
    B, N, _ = f1.shape

    def body(f1_ref, c2_ref, gm_ref, wa_ref, wb_ref, b3_ref, out_ref):
        f2 = jnp.maximum(c2_ref[0] + gm_ref[0], 0.0)
        pf = jnp.maximum(
            jnp.dot(f1_ref[0], wa_ref[...]) + jnp.dot(f2, wb_ref[...]) + b3_ref[...],
            0.0,
        )
        bmax = jnp.max(pf, axis=0, keepdims=True)

        @pl.when(pl.program_id(1) == 0)
        def _():
            out_ref[0] = bmax

        @pl.when(pl.program_id(1) > 0)
        def _():
            out_ref[0] = jnp.maximum(out_ref[0], bmax)

    return pl.pallas_call(
        body,
        grid=(B, N // _RM),
        in_specs=[
            pl.BlockSpec((1, _RM, 64), lambda b, r: (b, r, 0)),
            pl.BlockSpec((1, _RM, 128), lambda b, r: (b, r, 0)),
            pl.BlockSpec((1, _RM, 128), lambda b, r: (b, r, 0)),
            pl.BlockSpec((64, 256), lambda b, r: (0, 0)),
            pl.BlockSpec((128, 256), lambda b, r: (0, 0)),
            pl.BlockSpec((1, 256), lambda b, r: (0, 0)),
        ],
        out_specs=pl.BlockSpec((1, 1, 256), lambda b, r: (b, 0, 0)),
        out_shape=jax.ShapeDtypeStruct((B, 1, 256), jnp.float32),
    )(f1, c2, gm2, w3a, w3b, b3).reshape(B, 256)


def _decoder(pooled, w4, b4, w5, b5, w6, b6):
    B = pooled.shape[0]

    def body(p_ref, w4_ref, b4_ref, w5_ref, b5_ref, w6_ref, b6_ref,
             lab_ref, conf_ref):
        h = jnp.maximum(jnp.dot(p_ref[...], w4_ref[...]) + b4_ref[...], 0.0)
        h = jnp.maximum(jnp.dot(h, w5_ref[...]) + b5_ref[...], 0.0)
        logits = jnp.dot(h, w6_ref[...]) + b6_ref[...]
        m = jnp.max(logits, axis=1, keepdims=True)
        cls = lax.broadcasted_iota(jnp.int32, logits.shape, 1)
        lab_ref[...] = jnp.min(
            jnp.where(logits == m, cls, jnp.int32(_NCLS)), axis=1, keepdims=True)
        conf_ref[...] = 1.0 / jnp.sum(jnp.exp(logits - m), axis=1, keepdims=True)

    return pl.pallas_call(
        body,
        out_shape=[
            jax.ShapeDtypeStruct((B, 1), jnp.int32),
            jax.ShapeDtypeStruct((B, 1), jnp.float32),
        ],
    )(pooled, w4, b4, w5, b5, w6, b6)


def kernel(x, W1, b1, W2, b2, W3, b3, W4, b4, W5, b5, W6, b6):
    B, N, _ = x.shape
    M = B * N
    xt = jnp.transpose(x, (0, 2, 1))
    idx, c1, g1 = _knn_conv1_call(
        x, xt, W1[:3] - W1[3:], W1[3:], b1.reshape(1, -1))
    idx_flat = idx.reshape(M * _K)
    gm1 = _gather_max(g1.reshape(M, 128), idx_flat)
    f1, c2, g2 = _conv2_linears(
        c1.reshape(M, 64), gm1, W2[:64] - W2[64:], W2[64:], b2.reshape(1, -1))
    gm2 = _gather_max(g2, idx_flat)
    pooled = _head_pool(
        f1.reshape(B, N, 64), c2.reshape(B, N, 128), gm2.reshape(B, N, 128),
        W3[:64], W3[64:], b3.reshape(1, -1))
    lab, conf = _decoder(
        pooled, W4, b4.reshape(1, -1), W5, b5.reshape(1, -1), W6,
        b6.reshape(1, -1))
    labels = jnp.broadcast_to(lab, (B, N))
    confidences = jnp.broadcast_to(conf, (B, N))
    return labels, confidences
